# Initial kernel scaffold; baseline (speedup 1.0000x reference)
import jax
import jax.numpy as jnp
from jax.experimental import pallas as pl


def kernel(x, edge_index, W1, att_src1, att_dst1, b1, W2, att_src2, att_dst2, b2):
    raise NotImplementedError("write your pallas kernel here")


# speedup vs baseline: 22.4415x; 22.4415x over previous
import functools

import jax
import jax.numpy as jnp
from jax import lax
from jax.experimental import pallas as pl
from jax.experimental.pallas import tpu as pltpu
from jax.experimental.pallas import tpu_sc as plsc

N = 10000
E = 320000
F = 128
NPAD = 10240
NW = 32
EP = E // NW
B = 80
NB = EP // B
RPT = NPAD // 16

_f32 = jnp.float32
_i32 = jnp.int32


def _sc_edge_body(src_hbm, dst_hbm, h_hbm, asrc_hbm, adst_hbm,
                  acc_out, den_out,
                  asrc_v, adst_v, sidx_v, didx_v, rows_v, p_v, den_v,
                  zrow_v, acc_sh, sem):
    cid = lax.axis_index("c")
    sid = lax.axis_index("s")
    wid = cid * 16 + sid
    ebase = wid * EP

    zeros16 = jnp.zeros((16,), _f32)
    for i in range(16):
        for c in range(F // 16):
            zrow_v[i, pl.ds(c * 16, 16)] = zeros16

    def _zden(i, carry):
        den_v[pl.ds(i * 16, 16)] = zeros16
        return carry
    lax.fori_loop(0, NPAD // 16, _zden, 0)

    def _zacc(i, carry):
        pltpu.sync_copy(zrow_v, acc_sh.at[pl.ds(sid * RPT + i * 16, 16)])
        return carry
    lax.fori_loop(0, RPT // 16, _zacc, 0)

    pltpu.sync_copy(asrc_hbm, asrc_v)
    pltpu.sync_copy(adst_hbm, adst_v)

    plsc.subcore_barrier()

    def _block(j, carry):
        off = ebase + j * B
        pltpu.sync_copy(src_hbm.at[pl.ds(off, B)], sidx_v)
        pltpu.sync_copy(dst_hbm.at[pl.ds(off, B)], didx_v)
        cp = pltpu.async_copy(h_hbm.at[sidx_v], rows_v, sem)

        def _p16(k, c2):
            s = sidx_v[pl.ds(k * 16, 16)]
            d = didx_v[pl.ds(k * 16, 16)]
            a = plsc.load_gather(asrc_v, [s]) + plsc.load_gather(adst_v, [d])
            e = jnp.where(a >= 0.0, a, 0.2 * a)
            p = jnp.exp(e)
            p_v[pl.ds(k * 16, 16)] = p
            plsc.addupdate_scatter(den_v, [d], p)
            return c2
        lax.fori_loop(0, B // 16, _p16, 0)

        cp.wait()

        def _scale(i, c2):
            pv = plsc.load_gather(p_v, [jnp.full((16,), i, _i32)])
            for c in range(F // 16):
                rows_v[i, pl.ds(c * 16, 16)] = rows_v[i, pl.ds(c * 16, 16)] * pv
            return c2
        lax.fori_loop(0, B, _scale, 0)

        pltpu.sync_copy(rows_v, acc_sh.at[didx_v], add=True)
        return carry
    lax.fori_loop(0, NB, _block, 0)

    plsc.subcore_barrier()

    pltpu.sync_copy(acc_sh.at[pl.ds(sid * RPT, RPT)],
                    acc_out.at[cid, pl.ds(sid * RPT, RPT)])
    pltpu.sync_copy(den_v, den_out.at[cid, sid])


_sc_edges = pl.kernel(
    _sc_edge_body,
    out_type=[jax.ShapeDtypeStruct((2, NPAD, F), _f32),
              jax.ShapeDtypeStruct((2, 16, NPAD), _f32)],
    mesh=plsc.VectorSubcoreMesh(core_axis_name="c", subcore_axis_name="s"),
    compiler_params=pltpu.CompilerParams(needs_layout_passes=False),
    scratch_types=[
        pltpu.VMEM((NPAD,), _f32),
        pltpu.VMEM((NPAD,), _f32),
        pltpu.VMEM((B,), _i32),
        pltpu.VMEM((B,), _i32),
        pltpu.VMEM((B, F), _f32),
        pltpu.VMEM((B,), _f32),
        pltpu.VMEM((NPAD,), _f32),
        pltpu.VMEM((16, F), _f32),
        pltpu.VMEM_SHARED((NPAD, F), _f32),
        pltpu.SemaphoreType.DMA,
    ],
)


BR = 1280


def _tc_prep_body(x_ref, w_ref, avs_ref, avd_ref, h_ref, as_ref, ad_ref):
    h = jnp.dot(x_ref[...], w_ref[...], preferred_element_type=_f32)
    h_ref[...] = h
    as_ref[...] = jnp.dot(h, avs_ref[...], preferred_element_type=_f32)
    ad_ref[...] = jnp.dot(h, avd_ref[...], preferred_element_type=_f32)


_tc_prep = pl.pallas_call(
    _tc_prep_body,
    grid=(NPAD // BR,),
    in_specs=[
        pl.BlockSpec((BR, F), lambda i: (i, 0)),
        pl.BlockSpec((F, F), lambda i: (0, 0)),
        pl.BlockSpec((F, 1), lambda i: (0, 0)),
        pl.BlockSpec((F, 1), lambda i: (0, 0)),
    ],
    out_specs=[
        pl.BlockSpec((BR, F), lambda i: (i, 0)),
        pl.BlockSpec((BR, 1), lambda i: (i, 0)),
        pl.BlockSpec((BR, 1), lambda i: (i, 0)),
    ],
    out_shape=[jax.ShapeDtypeStruct((NPAD, F), _f32),
               jax.ShapeDtypeStruct((NPAD, 1), _f32),
               jax.ShapeDtypeStruct((NPAD, 1), _f32)],
)


def _tc_mid_body(acc_ref, den_ref, b_ref, w_ref, avs_ref, avd_ref,
                 h_ref, as_ref, ad_ref):
    den = jnp.sum(den_ref[...], axis=(0, 1))
    accsum = acc_ref[0] + acc_ref[1]
    g = accsum / (den[:, None] + 1e-16) + b_ref[...]
    g = jnp.maximum(g, 0.0)
    h = jnp.dot(g, w_ref[...], preferred_element_type=_f32)
    h_ref[...] = h
    as_ref[...] = jnp.dot(h, avs_ref[...], preferred_element_type=_f32)
    ad_ref[...] = jnp.dot(h, avd_ref[...], preferred_element_type=_f32)


_tc_mid = pl.pallas_call(
    _tc_mid_body,
    grid=(NPAD // BR,),
    in_specs=[
        pl.BlockSpec((2, BR, F), lambda i: (0, i, 0)),
        pl.BlockSpec((2, 16, BR), lambda i: (0, 0, i)),
        pl.BlockSpec((1, F), lambda i: (0, 0)),
        pl.BlockSpec((F, F), lambda i: (0, 0)),
        pl.BlockSpec((F, 1), lambda i: (0, 0)),
        pl.BlockSpec((F, 1), lambda i: (0, 0)),
    ],
    out_specs=[
        pl.BlockSpec((BR, F), lambda i: (i, 0)),
        pl.BlockSpec((BR, 1), lambda i: (i, 0)),
        pl.BlockSpec((BR, 1), lambda i: (i, 0)),
    ],
    out_shape=[jax.ShapeDtypeStruct((NPAD, F), _f32),
               jax.ShapeDtypeStruct((NPAD, 1), _f32),
               jax.ShapeDtypeStruct((NPAD, 1), _f32)],
)


def _tc_fin_body(acc_ref, den_ref, b_ref, out_ref):
    den = jnp.sum(den_ref[...], axis=(0, 1))
    out_ref[...] = (acc_ref[0] + acc_ref[1]) / (den[:, None] + 1e-16) + b_ref[...]


_tc_fin = pl.pallas_call(
    _tc_fin_body,
    grid=(NPAD // BR,),
    in_specs=[
        pl.BlockSpec((2, BR, F), lambda i: (0, i, 0)),
        pl.BlockSpec((2, 16, BR), lambda i: (0, 0, i)),
        pl.BlockSpec((1, F), lambda i: (0, 0)),
    ],
    out_specs=pl.BlockSpec((BR, F), lambda i: (i, 0)),
    out_shape=jax.ShapeDtypeStruct((NPAD, F), _f32),
)


def kernel(x, edge_index, W1, att_src1, att_dst1, b1, W2, att_src2,
           att_dst2, b2):
    src = edge_index[0].astype(_i32)
    dst = edge_index[1].astype(_i32)
    xp = jnp.pad(x, ((0, NPAD - N), (0, 0)))

    h1, as1, ad1 = _tc_prep(xp, W1, att_src1.reshape(F, 1),
                            att_dst1.reshape(F, 1))
    acc1, den1 = _sc_edges(src, dst, h1, as1.reshape(-1), ad1.reshape(-1))
    h2, as2, ad2 = _tc_mid(acc1, den1, b1.reshape(1, F), W2,
                           att_src2.reshape(F, 1), att_dst2.reshape(F, 1))
    acc2, den2 = _sc_edges(src, dst, h2, as2.reshape(-1), ad2.reshape(-1))
    out = _tc_fin(acc2, den2, b2.reshape(1, F))
    return out[:N]


# speedup vs baseline: 36.0214x; 1.6051x over previous
import functools

import jax
import jax.numpy as jnp
from jax import lax
from jax.experimental import pallas as pl
from jax.experimental.pallas import tpu as pltpu
from jax.experimental.pallas import tpu_sc as plsc

N = 10000
E = 320000
F = 128
NPAD = 10240
NW = 32
EP = E // NW
B = 80
NB = EP // B
RPT = NPAD // 16

_f32 = jnp.float32
_i32 = jnp.int32


def _sc_edge_body(src_hbm, dst_hbm, h_hbm, asrc_hbm, adst_hbm,
                  acc_out, den_out,
                  asrc_v, adst_v, sidx, didx, p_v, rows, zden_v,
                  acc_sh, den_sh, semi, semg):
    cid = lax.axis_index("c")
    sid = lax.axis_index("s")
    wid = cid * 16 + sid
    ebase = wid * EP

    zeros16 = jnp.zeros((16,), _f32)

    def _zrow(i, carry):
        for c in range(F // 16):
            rows[0][i, pl.ds(c * 16, 16)] = zeros16
        return carry
    lax.fori_loop(0, B, _zrow, 0)

    def _zd(i, carry):
        zden_v[pl.ds(i * 16, 16)] = zeros16
        return carry
    lax.fori_loop(0, RPT // 16, _zd, 0)

    def _zacc(i, carry):
        pltpu.sync_copy(rows[0], acc_sh.at[pl.ds(sid * RPT + i * B, B)])
        return carry
    lax.fori_loop(0, RPT // B, _zacc, 0)
    pltpu.sync_copy(zden_v, den_sh.at[pl.ds(sid * RPT, RPT)])

    pltpu.sync_copy(asrc_hbm.at[pl.ds(0, N)], asrc_v)
    pltpu.sync_copy(adst_hbm.at[pl.ds(0, N)], adst_v)

    plsc.subcore_barrier()

    def _idx_start(j, b):
        pltpu.async_copy(src_hbm.at[pl.ds(ebase + j * B, B)], sidx[b],
                         semi[b])
        pltpu.async_copy(dst_hbm.at[pl.ds(ebase + j * B, B)], didx[b],
                         semi[b])

    def _idx_wait(j, b):
        pltpu.make_async_copy(src_hbm.at[pl.ds(ebase + j * B, B)], sidx[b],
                              semi[b]).wait()
        pltpu.make_async_copy(dst_hbm.at[pl.ds(ebase + j * B, B)], didx[b],
                              semi[b]).wait()

    def _gather_start(b):
        pltpu.async_copy(h_hbm.at[sidx[b]], rows[b], semg[b])

    def _gather_wait(b):
        pltpu.make_async_copy(h_hbm.at[sidx[b]], rows[b], semg[b]).wait()

    def _process(b):
        def _pk(k, c2):
            s = sidx[b][pl.ds(k * 16, 16)]
            d = didx[b][pl.ds(k * 16, 16)]
            a = plsc.load_gather(asrc_v, [s]) + plsc.load_gather(adst_v, [d])
            e = jnp.where(a >= 0.0, a, 0.2 * a)
            p_v[b][pl.ds(k * 16, 16)] = jnp.exp(e)
            return c2
        lax.fori_loop(0, B // 16, _pk, 0)
        pltpu.sync_copy(p_v[b], den_sh.at[didx[b]], add=True)
        _gather_wait(b)

        def _srow(i, c2):
            pv = plsc.load_gather(p_v[b], [jnp.full((16,), i, _i32)])
            for c in range(F // 16):
                rows[b][i, pl.ds(c * 16, 16)] = (
                    rows[b][i, pl.ds(c * 16, 16)] * pv)
            return c2
        lax.fori_loop(0, B, _srow, 0, unroll=4)
        pltpu.sync_copy(rows[b], acc_sh.at[didx[b]], add=True)

    _idx_start(0, 0)
    _idx_wait(0, 0)
    _gather_start(0)
    _idx_start(1, 1)

    def _pair(j2, carry):
        j0 = 2 * j2
        _idx_wait(j0 + 1, 1)
        _gather_start(1)
        _process(0)
        _idx_start(j0 + 2, 0)
        _idx_wait(j0 + 2, 0)
        _gather_start(0)
        _process(1)
        _idx_start(j0 + 3, 1)
        return carry
    lax.fori_loop(0, (NB - 1) // 2 - 1, _pair, 0)

    _idx_wait(NB - 2, 1)
    _gather_start(1)
    _process(0)
    _idx_start(NB - 1, 0)
    _idx_wait(NB - 1, 0)
    _gather_start(0)
    _process(1)
    _process(0)

    plsc.subcore_barrier()

    pltpu.sync_copy(acc_sh.at[pl.ds(sid * RPT, RPT)],
                    acc_out.at[cid, pl.ds(sid * RPT, RPT)])
    pltpu.sync_copy(den_sh.at[pl.ds(sid * RPT, RPT)],
                    den_out.at[cid, pl.ds(sid * RPT, RPT)])


_sc_edges = pl.kernel(
    _sc_edge_body,
    out_type=[jax.ShapeDtypeStruct((2, NPAD, F), _f32),
              jax.ShapeDtypeStruct((2, NPAD), _f32)],
    mesh=plsc.VectorSubcoreMesh(core_axis_name="c", subcore_axis_name="s"),
    compiler_params=pltpu.CompilerParams(needs_layout_passes=False),
    scratch_types=[
        pltpu.VMEM((N,), _f32),
        pltpu.VMEM((N,), _f32),
        [pltpu.VMEM((B,), _i32)] * 2,
        [pltpu.VMEM((B,), _i32)] * 2,
        [pltpu.VMEM((B,), _f32)] * 2,
        [pltpu.VMEM((B, F), _f32)] * 2,
        pltpu.VMEM((RPT,), _f32),
        pltpu.VMEM_SHARED((NPAD, F), _f32),
        pltpu.VMEM_SHARED((NPAD,), _f32),
        [pltpu.SemaphoreType.DMA] * 2,
        [pltpu.SemaphoreType.DMA] * 2,
    ],
)


BR = 1280


def _tc_prep_body(x_ref, w_ref, avs_ref, avd_ref, h_ref, as_ref, ad_ref):
    h = jnp.dot(x_ref[...], w_ref[...], preferred_element_type=_f32)
    h_ref[...] = h
    as_ref[...] = jnp.dot(h, avs_ref[...], preferred_element_type=_f32)
    ad_ref[...] = jnp.dot(h, avd_ref[...], preferred_element_type=_f32)


_tc_prep = pl.pallas_call(
    _tc_prep_body,
    grid=(NPAD // BR,),
    in_specs=[
        pl.BlockSpec((BR, F), lambda i: (i, 0)),
        pl.BlockSpec((F, F), lambda i: (0, 0)),
        pl.BlockSpec((F, 1), lambda i: (0, 0)),
        pl.BlockSpec((F, 1), lambda i: (0, 0)),
    ],
    out_specs=[
        pl.BlockSpec((BR, F), lambda i: (i, 0)),
        pl.BlockSpec((BR, 1), lambda i: (i, 0)),
        pl.BlockSpec((BR, 1), lambda i: (i, 0)),
    ],
    out_shape=[jax.ShapeDtypeStruct((NPAD, F), _f32),
               jax.ShapeDtypeStruct((NPAD, 1), _f32),
               jax.ShapeDtypeStruct((NPAD, 1), _f32)],
)


def _tc_mid_body(acc_ref, den_ref, b_ref, w_ref, avs_ref, avd_ref,
                 h_ref, as_ref, ad_ref):
    den = den_ref[0] + den_ref[1]
    accsum = acc_ref[0] + acc_ref[1]
    g = accsum / (den[:, None] + 1e-16) + b_ref[...]
    g = jnp.maximum(g, 0.0)
    h = jnp.dot(g, w_ref[...], preferred_element_type=_f32)
    h_ref[...] = h
    as_ref[...] = jnp.dot(h, avs_ref[...], preferred_element_type=_f32)
    ad_ref[...] = jnp.dot(h, avd_ref[...], preferred_element_type=_f32)


_tc_mid = pl.pallas_call(
    _tc_mid_body,
    grid=(NPAD // BR,),
    in_specs=[
        pl.BlockSpec((2, BR, F), lambda i: (0, i, 0)),
        pl.BlockSpec((2, BR), lambda i: (0, i)),
        pl.BlockSpec((1, F), lambda i: (0, 0)),
        pl.BlockSpec((F, F), lambda i: (0, 0)),
        pl.BlockSpec((F, 1), lambda i: (0, 0)),
        pl.BlockSpec((F, 1), lambda i: (0, 0)),
    ],
    out_specs=[
        pl.BlockSpec((BR, F), lambda i: (i, 0)),
        pl.BlockSpec((BR, 1), lambda i: (i, 0)),
        pl.BlockSpec((BR, 1), lambda i: (i, 0)),
    ],
    out_shape=[jax.ShapeDtypeStruct((NPAD, F), _f32),
               jax.ShapeDtypeStruct((NPAD, 1), _f32),
               jax.ShapeDtypeStruct((NPAD, 1), _f32)],
)


def _tc_fin_body(acc_ref, den_ref, b_ref, out_ref):
    den = den_ref[0] + den_ref[1]
    out_ref[...] = (acc_ref[0] + acc_ref[1]) / (den[:, None] + 1e-16) + b_ref[...]


_tc_fin = pl.pallas_call(
    _tc_fin_body,
    grid=(NPAD // BR,),
    in_specs=[
        pl.BlockSpec((2, BR, F), lambda i: (0, i, 0)),
        pl.BlockSpec((2, BR), lambda i: (0, i)),
        pl.BlockSpec((1, F), lambda i: (0, 0)),
    ],
    out_specs=pl.BlockSpec((BR, F), lambda i: (i, 0)),
    out_shape=jax.ShapeDtypeStruct((NPAD, F), _f32),
)


def kernel(x, edge_index, W1, att_src1, att_dst1, b1, W2, att_src2,
           att_dst2, b2):
    src = edge_index[0].astype(_i32)
    dst = edge_index[1].astype(_i32)
    xp = jnp.pad(x, ((0, NPAD - N), (0, 0)))

    h1, as1, ad1 = _tc_prep(xp, W1, att_src1.reshape(F, 1),
                            att_dst1.reshape(F, 1))
    acc1, den1 = _sc_edges(src, dst, h1, as1.reshape(-1), ad1.reshape(-1))
    h2, as2, ad2 = _tc_mid(acc1, den1, b1.reshape(1, F), W2,
                           att_src2.reshape(F, 1), att_dst2.reshape(F, 1))
    acc2, den2 = _sc_edges(src, dst, h2, as2.reshape(-1), ad2.reshape(-1))
    out = _tc_fin(acc2, den2, b2.reshape(1, F))
    return out[:N]


# speedup vs baseline: 45.5110x; 1.2634x over previous
import functools

import jax
import jax.numpy as jnp
from jax import lax
from jax.experimental import pallas as pl
from jax.experimental.pallas import tpu as pltpu
from jax.experimental.pallas import tpu_sc as plsc

N = 10000
E = 320000
F = 128
NPAD = 10240
NW = 32
EP = E // NW
B = 80
NB = EP // B
RPT = NPAD // 16

_f32 = jnp.float32
_i32 = jnp.int32


def _sc_edge_body(src_hbm, dst_hbm, h_hbm, asrc_hbm, adst_hbm,
                  acc_out, den_out,
                  sidx, didx, asv, adv, p_v, dsc, rows, zden_v,
                  acc_sh, den_sh, semi, semv, sems):
    cid = lax.axis_index("c")
    sid = lax.axis_index("s")
    wid = cid * 16 + sid
    ebase = wid * EP

    zeros16 = jnp.zeros((16,), _f32)

    def _zrow(i, carry):
        for c in range(F // 16):
            rows[0][i, pl.ds(c * 16, 16)] = zeros16
        return carry
    lax.fori_loop(0, B, _zrow, 0)

    def _zd(i, carry):
        zden_v[pl.ds(i * 16, 16)] = zeros16
        return carry
    lax.fori_loop(0, RPT // 16, _zd, 0)

    def _zacc(i, carry):
        pltpu.sync_copy(rows[0], acc_sh.at[pl.ds(sid * RPT + i * B, B)])
        return carry
    lax.fori_loop(0, RPT // B, _zacc, 0)
    pltpu.sync_copy(zden_v, den_sh.at[pl.ds(sid * RPT, RPT)])

    def _idx_start(j, k):
        off = ebase + j * B
        pltpu.async_copy(src_hbm.at[pl.ds(off, B)], sidx[k], semi[k])
        pltpu.async_copy(dst_hbm.at[pl.ds(off, B)], didx[k], semi[k])

    def _idx_wait(j, k):
        off = ebase + j * B
        pltpu.make_async_copy(src_hbm.at[pl.ds(off, B)], sidx[k],
                              semi[k]).wait()
        pltpu.make_async_copy(dst_hbm.at[pl.ds(off, B)], didx[k],
                              semi[k]).wait()

    def _val_start(k):
        pltpu.async_copy(asrc_hbm.at[sidx[k]], asv[k], semv[k])
        pltpu.async_copy(adst_hbm.at[didx[k]], adv[k], semv[k])
        pltpu.async_copy(h_hbm.at[sidx[k]], rows[k], semv[k])

    def _val_wait(k):
        pltpu.make_async_copy(asrc_hbm.at[sidx[k]], asv[k], semv[k]).wait()
        pltpu.make_async_copy(adst_hbm.at[didx[k]], adv[k], semv[k]).wait()
        pltpu.make_async_copy(h_hbm.at[sidx[k]], rows[k], semv[k]).wait()

    def _phase1(k):
        for q in range(B // 16):
            a = asv[k][pl.ds(q * 16, 16)] + adv[k][pl.ds(q * 16, 16)]
            e = jnp.where(a >= 0.0, a, 0.2 * a)
            p_v[k][pl.ds(q * 16, 16)] = jnp.exp(e)
            dsc[k][pl.ds(q * 16, 16)] = didx[k][pl.ds(q * 16, 16)]
        pltpu.async_copy(p_v[k], den_sh.at[dsc[k]], sems[k], add=True)

    def _phase2(k):
        def _srow(i, c2):
            pv = plsc.load_gather(p_v[k], [jnp.full((16,), i, _i32)])
            for c in range(F // 16):
                rows[k][i, pl.ds(c * 16, 16)] = (
                    rows[k][i, pl.ds(c * 16, 16)] * pv)
            return c2
        lax.fori_loop(0, B, _srow, 0, unroll=4)
        pltpu.async_copy(rows[k], acc_sh.at[dsc[k]], sems[k], add=True)

    def _scwait(k):
        pltpu.make_async_copy(p_v[k], den_sh.at[dsc[k]], sems[k]).wait()
        pltpu.make_async_copy(rows[k], acc_sh.at[dsc[k]], sems[k]).wait()

    _idx_start(0, 0)
    _idx_wait(0, 0)
    _val_start(0)
    _idx_start(1, 1)

    plsc.subcore_barrier()

    _val_wait(0)
    _phase1(0)
    _idx_wait(1, 1)
    _val_start(1)
    _phase2(0)
    _idx_start(2, 2)
    _val_wait(1)
    _phase1(1)
    _idx_wait(2, 2)
    _val_start(2)
    _phase2(1)
    _idx_start(3, 0)

    def _grp(g, carry):
        jb = 3 * g + 2
        for t in range(3):
            j = jb + t
            k = (2 + t) % 3
            _val_wait(k)
            _phase1(k)
            _scwait((k + 1) % 3)
            _idx_wait(j + 1, (k + 1) % 3)
            _val_start((k + 1) % 3)
            _phase2(k)
            _idx_start(j + 2, (k + 2) % 3)
        return carry
    lax.fori_loop(0, 40, _grp, 0)

    _val_wait(2)
    _phase1(2)
    _scwait(0)
    _idx_wait(123, 0)
    _val_start(0)
    _phase2(2)
    _idx_start(124, 1)

    _val_wait(0)
    _phase1(0)
    _scwait(1)
    _idx_wait(124, 1)
    _val_start(1)
    _phase2(0)

    _val_wait(1)
    _phase1(1)
    _phase2(1)

    _scwait(2)
    _scwait(0)
    _scwait(1)

    plsc.subcore_barrier()

    pltpu.sync_copy(acc_sh.at[pl.ds(sid * RPT, RPT)],
                    acc_out.at[cid, pl.ds(sid * RPT, RPT)])
    pltpu.sync_copy(den_sh.at[pl.ds(sid * RPT, RPT)],
                    den_out.at[cid, pl.ds(sid * RPT, RPT)])


_sc_edges = pl.kernel(
    _sc_edge_body,
    out_type=[jax.ShapeDtypeStruct((2, NPAD, F), _f32),
              jax.ShapeDtypeStruct((2, NPAD), _f32)],
    mesh=plsc.VectorSubcoreMesh(core_axis_name="c", subcore_axis_name="s"),
    compiler_params=pltpu.CompilerParams(needs_layout_passes=False),
    scratch_types=[
        [pltpu.VMEM((B,), _i32)] * 3,
        [pltpu.VMEM((B,), _i32)] * 3,
        [pltpu.VMEM((B,), _f32)] * 3,
        [pltpu.VMEM((B,), _f32)] * 3,
        [pltpu.VMEM((B,), _f32)] * 3,
        [pltpu.VMEM((B,), _i32)] * 3,
        [pltpu.VMEM((B, F), _f32)] * 3,
        pltpu.VMEM((RPT,), _f32),
        pltpu.VMEM_SHARED((NPAD, F), _f32),
        pltpu.VMEM_SHARED((NPAD,), _f32),
        [pltpu.SemaphoreType.DMA] * 3,
        [pltpu.SemaphoreType.DMA] * 3,
        [pltpu.SemaphoreType.DMA] * 3,
    ],
)


BR = 1280


def _tc_prep_body(x_ref, w_ref, avs_ref, avd_ref, h_ref, as_ref, ad_ref):
    h = jnp.dot(x_ref[...], w_ref[...], preferred_element_type=_f32)
    h_ref[...] = h
    as_ref[...] = jnp.dot(h, avs_ref[...], preferred_element_type=_f32)
    ad_ref[...] = jnp.dot(h, avd_ref[...], preferred_element_type=_f32)


_tc_prep = pl.pallas_call(
    _tc_prep_body,
    grid=(NPAD // BR,),
    in_specs=[
        pl.BlockSpec((BR, F), lambda i: (i, 0)),
        pl.BlockSpec((F, F), lambda i: (0, 0)),
        pl.BlockSpec((F, 1), lambda i: (0, 0)),
        pl.BlockSpec((F, 1), lambda i: (0, 0)),
    ],
    out_specs=[
        pl.BlockSpec((BR, F), lambda i: (i, 0)),
        pl.BlockSpec((BR, 1), lambda i: (i, 0)),
        pl.BlockSpec((BR, 1), lambda i: (i, 0)),
    ],
    out_shape=[jax.ShapeDtypeStruct((NPAD, F), _f32),
               jax.ShapeDtypeStruct((NPAD, 1), _f32),
               jax.ShapeDtypeStruct((NPAD, 1), _f32)],
)


def _tc_mid_body(acc_ref, den_ref, b_ref, w_ref, avs_ref, avd_ref,
                 h_ref, as_ref, ad_ref):
    den = den_ref[0] + den_ref[1]
    accsum = acc_ref[0] + acc_ref[1]
    g = accsum / (den[:, None] + 1e-16) + b_ref[...]
    g = jnp.maximum(g, 0.0)
    h = jnp.dot(g, w_ref[...], preferred_element_type=_f32)
    h_ref[...] = h
    as_ref[...] = jnp.dot(h, avs_ref[...], preferred_element_type=_f32)
    ad_ref[...] = jnp.dot(h, avd_ref[...], preferred_element_type=_f32)


_tc_mid = pl.pallas_call(
    _tc_mid_body,
    grid=(NPAD // BR,),
    in_specs=[
        pl.BlockSpec((2, BR, F), lambda i: (0, i, 0)),
        pl.BlockSpec((2, BR), lambda i: (0, i)),
        pl.BlockSpec((1, F), lambda i: (0, 0)),
        pl.BlockSpec((F, F), lambda i: (0, 0)),
        pl.BlockSpec((F, 1), lambda i: (0, 0)),
        pl.BlockSpec((F, 1), lambda i: (0, 0)),
    ],
    out_specs=[
        pl.BlockSpec((BR, F), lambda i: (i, 0)),
        pl.BlockSpec((BR, 1), lambda i: (i, 0)),
        pl.BlockSpec((BR, 1), lambda i: (i, 0)),
    ],
    out_shape=[jax.ShapeDtypeStruct((NPAD, F), _f32),
               jax.ShapeDtypeStruct((NPAD, 1), _f32),
               jax.ShapeDtypeStruct((NPAD, 1), _f32)],
)


def _tc_fin_body(acc_ref, den_ref, b_ref, out_ref):
    den = den_ref[0] + den_ref[1]
    out_ref[...] = (acc_ref[0] + acc_ref[1]) / (den[:, None] + 1e-16) + b_ref[...]


_tc_fin = pl.pallas_call(
    _tc_fin_body,
    grid=(NPAD // BR,),
    in_specs=[
        pl.BlockSpec((2, BR, F), lambda i: (0, i, 0)),
        pl.BlockSpec((2, BR), lambda i: (0, i)),
        pl.BlockSpec((1, F), lambda i: (0, 0)),
    ],
    out_specs=pl.BlockSpec((BR, F), lambda i: (i, 0)),
    out_shape=jax.ShapeDtypeStruct((NPAD, F), _f32),
)


def kernel(x, edge_index, W1, att_src1, att_dst1, b1, W2, att_src2,
           att_dst2, b2):
    src = edge_index[0].astype(_i32)
    dst = edge_index[1].astype(_i32)
    xp = jnp.pad(x, ((0, NPAD - N), (0, 0)))

    h1, as1, ad1 = _tc_prep(xp, W1, att_src1.reshape(F, 1),
                            att_dst1.reshape(F, 1))
    acc1, den1 = _sc_edges(src, dst, h1, as1.reshape(-1), ad1.reshape(-1))
    h2, as2, ad2 = _tc_mid(acc1, den1, b1.reshape(1, F), W2,
                           att_src2.reshape(F, 1), att_dst2.reshape(F, 1))
    acc2, den2 = _sc_edges(src, dst, h2, as2.reshape(-1), ad2.reshape(-1))
    out = _tc_fin(acc2, den2, b2.reshape(1, F))
    return out[:N]
